# baseline (speedup 1.0000x reference)
import functools

import jax
import jax.numpy as jnp
from jax import lax
from jax.experimental import pallas as pl
from jax.experimental.pallas import tpu as pltpu
from jax.experimental.pallas import tpu_sc as plsc

N_NODES = 10000
N_PAD = 10240
N_EDGES = 320000
BLK = 80
NBLKS = N_EDGES // BLK
NC = 2
NS = 16
ROWS_PER_TILE = N_PAD // NS
NBLK_TILE = NBLKS // NS
NBLK_C0 = 124
NBLK_C1 = NBLK_TILE - NBLK_C0

_mesh = plsc.VectorSubcoreMesh(
    core_axis_name="c", subcore_axis_name="s", num_cores=NC, num_subcores=NS
)
_sc_params = pltpu.CompilerParams(use_tc_tiling_on_sc=False)


@functools.partial(
    pl.kernel,
    out_type=jax.ShapeDtypeStruct((N_PAD,), jnp.float32),
    mesh=_mesh,
    scratch_types=[
        pltpu.VMEM((NBLK_TILE, BLK), jnp.int32),
        pltpu.VMEM((BLK,), jnp.float32),
        pltpu.VMEM((ROWS_PER_TILE,), jnp.float32),
        pltpu.VMEM_SHARED((N_PAD,), jnp.float32),
        pltpu.SemaphoreType.DMA,
    ],
    compiler_params=_sc_params,
)
def _deg_dinv_kernel(dst_hbm, zeros_hbm, dinv_hbm, idx_v, ones_v, buf_v, degacc, sem):
    c = lax.axis_index("c")
    s = lax.axis_index("s")

    @pl.when(c == 1)
    def _():
        base = s * ROWS_PER_TILE
        pltpu.sync_copy(
            zeros_hbm.at[pl.ds(base, ROWS_PER_TILE)],
            degacc.at[pl.ds(base, ROWS_PER_TILE)],
        )
        pltpu.sync_copy(dst_hbm.at[pl.ds(s * NBLK_TILE, NBLK_TILE)], idx_v)

        @pl.loop(0, BLK // 16)
        def _(i):
            ones_v[pl.ds(i * 16, 16)] = jnp.ones((16,), jnp.float32)

        plsc.subcore_barrier()

        @pl.loop(0, NBLK_TILE)
        def _(j):
            pltpu.async_copy(ones_v, degacc.at[idx_v.at[j]], sem, add=True)

        @pl.loop(0, NBLK_TILE)
        def _(j):
            pltpu.make_async_copy(ones_v, degacc.at[idx_v.at[j]], sem).wait()

        plsc.subcore_barrier()

        pltpu.sync_copy(degacc.at[pl.ds(base, ROWS_PER_TILE)], buf_v)

        @pl.loop(0, ROWS_PER_TILE // 16)
        def _(i):
            v = buf_v[pl.ds(i * 16, 16)] + 1.0
            iv = lax.bitcast_convert_type(v, jnp.int32)
            iv = jnp.int32(0x5F3759DF) - (iv >> 1)
            y = lax.bitcast_convert_type(iv, jnp.float32)
            y = y * (1.5 - 0.5 * v * y * y)
            y = y * (1.5 - 0.5 * v * y * y)
            y = y * (1.5 - 0.5 * v * y * y)
            buf_v[pl.ds(i * 16, 16)] = y

        pltpu.sync_copy(buf_v, dinv_hbm.at[pl.ds(base, ROWS_PER_TILE)])


def _make_agg_kernel(feat, dtype):
    @functools.partial(
        pl.kernel,
        out_type=jax.ShapeDtypeStruct((NC, N_PAD, feat), dtype),
        mesh=_mesh,
        scratch_types=[
            pltpu.VMEM((NBLK_C1, BLK), jnp.int32),
            pltpu.VMEM((NBLK_C1, BLK), jnp.int32),
            pltpu.VMEM((2, BLK, feat), dtype),
            pltpu.VMEM_SHARED((N_PAD, feat), dtype),
            pltpu.SemaphoreType.DMA,
            pltpu.SemaphoreType.DMA,
            pltpu.SemaphoreType.DMA,
            pltpu.SemaphoreType.DMA,
        ],
        compiler_params=_sc_params,
    )
    def _agg(g_hbm, src_hbm, dst_hbm, zeros_hbm, out_hbm, srcv, dstv, rows, acc,
             semg0, semg1, sems0, sems1):
        c = lax.axis_index("c")
        s = lax.axis_index("s")
        base = s * ROWS_PER_TILE
        pltpu.sync_copy(
            zeros_hbm.at[pl.ds(base, ROWS_PER_TILE)],
            acc.at[pl.ds(base, ROWS_PER_TILE)],
        )

        def run(start_blk, nblk):
            pltpu.sync_copy(
                src_hbm.at[pl.ds(start_blk, nblk)], srcv.at[pl.ds(0, nblk)]
            )
            pltpu.sync_copy(
                dst_hbm.at[pl.ds(start_blk, nblk)], dstv.at[pl.ds(0, nblk)]
            )
            pltpu.async_copy(g_hbm.at[srcv.at[0]], rows.at[0], semg0)
            pltpu.async_copy(g_hbm.at[srcv.at[1]], rows.at[1], semg1)

            @pl.loop(0, nblk, step=2)
            def _(j):
                pltpu.make_async_copy(g_hbm.at[srcv.at[j]], rows.at[0], semg0).wait()
                pltpu.async_copy(rows.at[0], acc.at[dstv.at[j]], sems0, add=True)
                pltpu.make_async_copy(
                    g_hbm.at[srcv.at[j + 1]], rows.at[1], semg1
                ).wait()
                pltpu.async_copy(rows.at[1], acc.at[dstv.at[j + 1]], sems1, add=True)
                pltpu.make_async_copy(rows.at[0], acc.at[dstv.at[j]], sems0).wait()

                @pl.when(j + 2 < nblk)
                def _():
                    pltpu.async_copy(g_hbm.at[srcv.at[j + 2]], rows.at[0], semg0)

                pltpu.make_async_copy(rows.at[1], acc.at[dstv.at[j + 1]], sems1).wait()

                @pl.when(j + 3 < nblk)
                def _():
                    pltpu.async_copy(g_hbm.at[srcv.at[j + 3]], rows.at[1], semg1)

        @pl.when(c == 0)
        def _():
            run(s * NBLK_TILE, NBLK_C0)

        @pl.when(c == 1)
        def _():
            run(s * NBLK_TILE + NBLK_C0, NBLK_C1)

        plsc.subcore_barrier()
        pltpu.sync_copy(
            acc.at[pl.ds(base, ROWS_PER_TILE)],
            out_hbm.at[c, pl.ds(base, ROWS_PER_TILE)],
        )

    return _agg


_agg32 = _make_agg_kernel(32, jnp.bfloat16)
_agg16 = _make_agg_kernel(16, jnp.float32)


TC_GRID = 5
TC_ROWS = N_NODES // TC_GRID


def _tc1_body(x_ref, w1_ref, dinv_ref, g1_ref, dinvb_ref):
    h = jnp.dot(x_ref[...], w1_ref[...], preferred_element_type=jnp.float32)
    d = dinv_ref[...]
    g1_ref[...] = (h * d).astype(jnp.bfloat16)
    dinvb_ref[...] = jnp.broadcast_to(d, (TC_ROWS, 32))


def _tc2_body(agg_ref, g1_ref, dinvb_ref, w2_ref, b1_ref, g2_ref):
    a = (agg_ref[0].astype(jnp.float32) + agg_ref[1].astype(jnp.float32)
         + g1_ref[...].astype(jnp.float32))
    out1 = jnp.maximum(a * dinvb_ref[...] + b1_ref[...], 0.0)
    h2 = jnp.dot(out1, w2_ref[...], preferred_element_type=jnp.float32)
    g2_ref[...] = h2 * dinvb_ref[:, :16]


def _tc3_body(agg_ref, g2_ref, dinvb_ref, wl_ref, b2_ref, bl_ref, out_ref):
    a = agg_ref[0] + agg_ref[1] + g2_ref[...]
    out2 = jnp.maximum(a * dinvb_ref[:, :16] + b2_ref[...], 0.0)
    out_ref[...] = (
        jnp.dot(out2, wl_ref[...], preferred_element_type=jnp.float32) + bl_ref[...]
    )


def _row_spec(cols, dtype=None):
    return pl.BlockSpec((TC_ROWS, cols), lambda i: (i, 0))


def _agg_spec(cols):
    return pl.BlockSpec((2, TC_ROWS, cols), lambda i: (0, i, 0))


def _whole_spec(shape):
    return pl.BlockSpec(shape, lambda i: tuple(0 for _ in shape))


def kernel(x, edge_index, W1, b1, W2, b2, Wl, bl):
    f32 = jnp.float32
    ei = edge_index.astype(jnp.int32)
    src3 = ei[0].reshape(NBLKS, BLK)
    dst3 = ei[1].reshape(NBLKS, BLK)
    zeros1 = jnp.zeros((N_PAD,), f32)
    zeros32 = jnp.zeros((N_PAD, 32), jnp.bfloat16)
    zeros16 = jnp.zeros((N_PAD, 16), f32)

    dinv = _deg_dinv_kernel(dst3, zeros1)
    dinv2d = dinv.reshape(N_PAD, 1)

    g1, dinvb = pl.pallas_call(
        _tc1_body,
        grid=(TC_GRID,),
        in_specs=[_row_spec(128), _whole_spec((128, 32)), _row_spec(1)],
        out_specs=[_row_spec(32), _row_spec(32)],
        out_shape=[
            jax.ShapeDtypeStruct((N_NODES, 32), jnp.bfloat16),
            jax.ShapeDtypeStruct((N_NODES, 32), f32),
        ],
    )(x, W1, dinv2d)

    agg1 = _agg32(g1, src3, dst3, zeros32)

    g2 = pl.pallas_call(
        _tc2_body,
        grid=(TC_GRID,),
        in_specs=[_agg_spec(32), _row_spec(32), _row_spec(32),
                  _whole_spec((32, 16)), _whole_spec((1, 32))],
        out_specs=_row_spec(16),
        out_shape=jax.ShapeDtypeStruct((N_NODES, 16), f32),
    )(agg1, g1, dinvb, W2, b1.reshape(1, 32))

    agg2 = _agg16(g2, src3, dst3, zeros16)

    out = pl.pallas_call(
        _tc3_body,
        grid=(TC_GRID,),
        in_specs=[_agg_spec(16), _row_spec(16), _row_spec(32),
                  _whole_spec((16, 1)), _whole_spec((1, 16)), _whole_spec((1, 1))],
        out_specs=_row_spec(1),
        out_shape=jax.ShapeDtypeStruct((N_NODES, 1), f32),
    )(agg2, g2, dinvb, Wl, b2.reshape(1, 16), bl.reshape(1, 1))

    return out


# speedup vs baseline: 1.0539x; 1.0539x over previous
import functools

import jax
import jax.numpy as jnp
from jax import lax
from jax.experimental import pallas as pl
from jax.experimental.pallas import tpu as pltpu
from jax.experimental.pallas import tpu_sc as plsc

N_NODES = 10000
N_PAD = 10240
N_EDGES = 320000
BLK = 80
NBLKS = N_EDGES // BLK
NC = 2
NS = 16
ROWS_PER_TILE = N_PAD // NS
NBLK_TILE = NBLKS // NS
NBLK_C0 = 124
NBLK_C1 = NBLK_TILE - NBLK_C0

_mesh = plsc.VectorSubcoreMesh(
    core_axis_name="c", subcore_axis_name="s", num_cores=NC, num_subcores=NS
)
_sc_params = pltpu.CompilerParams(use_tc_tiling_on_sc=False)


@functools.partial(
    pl.kernel,
    out_type=jax.ShapeDtypeStruct((N_PAD, 16), jnp.float32),
    mesh=_mesh,
    scratch_types=[
        pltpu.VMEM((NBLK_TILE, BLK), jnp.int32),
        pltpu.VMEM((BLK,), jnp.float32),
        pltpu.VMEM((ROWS_PER_TILE,), jnp.float32),
        pltpu.VMEM((ROWS_PER_TILE, 16), jnp.float32),
        pltpu.VMEM_SHARED((N_PAD,), jnp.float32),
        pltpu.SemaphoreType.DMA,
    ],
    compiler_params=_sc_params,
)
def _deg_dinv_kernel(dst_hbm, zeros_hbm, dinv_hbm, idx_v, ones_v, buf_v, row_v,
                     degacc, sem):
    c = lax.axis_index("c")
    s = lax.axis_index("s")

    @pl.when(c == 1)
    def _():
        base = s * ROWS_PER_TILE
        pltpu.sync_copy(
            zeros_hbm.at[pl.ds(base, ROWS_PER_TILE)],
            degacc.at[pl.ds(base, ROWS_PER_TILE)],
        )
        pltpu.sync_copy(dst_hbm.at[1, pl.ds(s * NBLK_TILE, NBLK_TILE)], idx_v)

        @pl.loop(0, BLK // 16)
        def _(i):
            ones_v[pl.ds(i * 16, 16)] = jnp.ones((16,), jnp.float32)

        plsc.subcore_barrier()

        @pl.loop(0, NBLK_TILE)
        def _(j):
            pltpu.async_copy(ones_v, degacc.at[idx_v.at[j]], sem, add=True)

        @pl.loop(0, NBLK_TILE)
        def _(j):
            pltpu.make_async_copy(ones_v, degacc.at[idx_v.at[j]], sem).wait()

        plsc.subcore_barrier()

        pltpu.sync_copy(degacc.at[pl.ds(base, ROWS_PER_TILE)], buf_v)

        @pl.loop(0, ROWS_PER_TILE // 16)
        def _(i):
            v = buf_v[pl.ds(i * 16, 16)] + 1.0
            iv = lax.bitcast_convert_type(v, jnp.int32)
            iv = jnp.int32(0x5F3759DF) - (iv >> 1)
            y = lax.bitcast_convert_type(iv, jnp.float32)
            y = y * (1.5 - 0.5 * v * y * y)
            y = y * (1.5 - 0.5 * v * y * y)
            y = y * (1.5 - 0.5 * v * y * y)
            for l in range(16):
                row_v[i * 16 + l] = jnp.full((16,), y[l], jnp.float32)

        pltpu.sync_copy(row_v, dinv_hbm.at[pl.ds(base, ROWS_PER_TILE), :])


def _make_agg_kernel(feat, dtype):
    @functools.partial(
        pl.kernel,
        out_type=jax.ShapeDtypeStruct((NC, N_PAD, feat), dtype),
        mesh=_mesh,
        scratch_types=[
            pltpu.VMEM((NBLK_C1, BLK), jnp.int32),
            pltpu.VMEM((NBLK_C1, BLK), jnp.int32),
            pltpu.VMEM((2, BLK, feat), dtype),
            pltpu.VMEM_SHARED((N_PAD, feat), dtype),
            pltpu.SemaphoreType.DMA,
            pltpu.SemaphoreType.DMA,
            pltpu.SemaphoreType.DMA,
            pltpu.SemaphoreType.DMA,
        ],
        compiler_params=_sc_params,
    )
    def _agg(g_hbm, e_hbm, zeros_hbm, out_hbm, srcv, dstv, rows, acc,
             semg0, semg1, sems0, sems1):
        c = lax.axis_index("c")
        s = lax.axis_index("s")
        base = s * ROWS_PER_TILE
        pltpu.sync_copy(
            zeros_hbm.at[pl.ds(base, ROWS_PER_TILE)],
            acc.at[pl.ds(base, ROWS_PER_TILE)],
        )

        def run(start_blk, nblk):
            pltpu.sync_copy(
                e_hbm.at[0, pl.ds(start_blk, nblk)], srcv.at[pl.ds(0, nblk)]
            )
            pltpu.sync_copy(
                e_hbm.at[1, pl.ds(start_blk, nblk)], dstv.at[pl.ds(0, nblk)]
            )
            pltpu.async_copy(g_hbm.at[srcv.at[0]], rows.at[0], semg0)
            pltpu.async_copy(g_hbm.at[srcv.at[1]], rows.at[1], semg1)

            @pl.loop(0, nblk, step=2)
            def _(j):
                pltpu.make_async_copy(g_hbm.at[srcv.at[j]], rows.at[0], semg0).wait()
                pltpu.async_copy(rows.at[0], acc.at[dstv.at[j]], sems0, add=True)
                pltpu.make_async_copy(
                    g_hbm.at[srcv.at[j + 1]], rows.at[1], semg1
                ).wait()
                pltpu.async_copy(rows.at[1], acc.at[dstv.at[j + 1]], sems1, add=True)
                pltpu.make_async_copy(rows.at[0], acc.at[dstv.at[j]], sems0).wait()

                @pl.when(j + 2 < nblk)
                def _():
                    pltpu.async_copy(g_hbm.at[srcv.at[j + 2]], rows.at[0], semg0)

                pltpu.make_async_copy(rows.at[1], acc.at[dstv.at[j + 1]], sems1).wait()

                @pl.when(j + 3 < nblk)
                def _():
                    pltpu.async_copy(g_hbm.at[srcv.at[j + 3]], rows.at[1], semg1)

        @pl.when(c == 0)
        def _():
            run(s * NBLK_TILE, NBLK_C0)

        @pl.when(c == 1)
        def _():
            run(s * NBLK_TILE + NBLK_C0, NBLK_C1)

        plsc.subcore_barrier()
        pltpu.sync_copy(
            acc.at[pl.ds(base, ROWS_PER_TILE)],
            out_hbm.at[c, pl.ds(base, ROWS_PER_TILE)],
        )

    return _agg


_agg32 = _make_agg_kernel(32, jnp.bfloat16)
_agg16 = _make_agg_kernel(16, jnp.float32)


def _tc1_body(x_ref, w1_ref, dinv_ref, g1_ref):
    h = jnp.dot(x_ref[...], w1_ref[...], preferred_element_type=jnp.float32)
    g1_ref[...] = (h * dinv_ref[:N_NODES, :1]).astype(jnp.bfloat16)


def _tc2_body(agg_ref, g1_ref, dinv_ref, w2_ref, b1_ref, g2_ref):
    d = dinv_ref[:N_NODES, :1]
    a = (agg_ref[0, :N_NODES].astype(jnp.float32)
         + agg_ref[1, :N_NODES].astype(jnp.float32)
         + g1_ref[...].astype(jnp.float32))
    out1 = jnp.maximum(a * d + b1_ref[...], 0.0)
    h2 = jnp.dot(out1, w2_ref[...], preferred_element_type=jnp.float32)
    g2_ref[...] = h2 * d


def _tc3_body(agg_ref, g2_ref, dinv_ref, wl_ref, b2_ref, bl_ref, out_ref):
    a = agg_ref[0, :N_NODES] + agg_ref[1, :N_NODES] + g2_ref[...]
    out2 = jnp.maximum(a * dinv_ref[:N_NODES, :1] + b2_ref[...], 0.0)
    out_ref[...] = (
        jnp.dot(out2, wl_ref[...], preferred_element_type=jnp.float32) + bl_ref[...]
    )


def kernel(x, edge_index, W1, b1, W2, b2, Wl, bl):
    f32 = jnp.float32
    e3 = edge_index.astype(jnp.int32).reshape(2, NBLKS, BLK)
    zeros1 = jnp.zeros((N_PAD,), f32)
    zeros32 = jnp.zeros((N_PAD, 32), jnp.bfloat16)
    zeros16 = jnp.zeros((N_PAD, 16), f32)

    dinv16 = _deg_dinv_kernel(e3, zeros1)

    g1 = pl.pallas_call(
        _tc1_body, out_shape=jax.ShapeDtypeStruct((N_NODES, 32), jnp.bfloat16)
    )(x, W1, dinv16)

    agg1 = _agg32(g1, e3, zeros32)

    g2 = pl.pallas_call(
        _tc2_body, out_shape=jax.ShapeDtypeStruct((N_NODES, 16), f32)
    )(agg1, g1, dinv16, W2, b1.reshape(1, 32))

    agg2 = _agg16(g2, e3, zeros16)

    out = pl.pallas_call(
        _tc3_body, out_shape=jax.ShapeDtypeStruct((N_NODES, 1), f32)
    )(agg2, g2, dinv16, Wl, b2.reshape(1, 16), bl.reshape(1, 1))

    return out
